# baseline (speedup 1.0000x reference)
import functools

import jax
import jax.numpy as jnp
from jax import lax
from jax.experimental import pallas as pl
from jax.experimental.pallas import tpu as pltpu
from jax.experimental.pallas import tpu_sc as plsc

B = 16384
DU = 64
DI = 32
FU = 128
FI = 128
FC = 128
NUM_TASKS = 4
CROSS_OUT = 128
VOCAB = 1000000

NC = 2
NS = 16
NW = NC * NS
BPW = B // NW
IDX_CHUNK = 128

VB = 32768
NBLK = -(-VOCAB // VB)
LB = VB // 128
NLINES = NBLK * LB
VFLAT = NLINES * 128

BB = 1024
NDB = B // BB


def _pack_body(utabT, itabT, wt, uf, itf, cf, wu, wi, wc, bu, bi, bc,
               *outs):
    au = wt[0:DU, :]
    ai = wt[2 * DU:2 * DU + DI, :]
    tn = (((0,), (0,)), ((), ()))
    pu = lax.dot_general(au, utabT[...], tn,
                         preferred_element_type=jnp.float32)
    pi = lax.dot_general(ai, itabT[...], tn,
                         preferred_element_type=jnp.float32)
    for t in range(NUM_TASKS):
        outs[t][...] = pu[t:t + 1, :].reshape(LB, 128)
        outs[NUM_TASKS + t][...] = pi[t:t + 1, :].reshape(LB, 128)

    f32 = jnp.float32
    dot = functools.partial(jnp.dot, preferred_element_type=f32)
    uft = dot(uf[...], wu[...]) + bu[...]
    ift = dot(itf[...], wi[...]) + bi[...]
    cft = dot(cf[...], wc[...]) + bc[...]
    wt_all = wt[...]
    acc = dot(uft, wt_all[DU:2 * DU, :])
    acc += dot(ift, wt_all[2 * DU + DI:2 * DU + 2 * DI, :])
    acc += dot(cft, wt_all[2 * DU + 2 * DI:, :])
    outs[8][...] = lax.transpose(acc, (1, 0))


def _pack_call(utabT, itabT, wt, uf, itf, cf, wu, wi, wc, bu, bi, bc):
    dense_i = lambda i: (jnp.minimum(i, NDB - 1), 0)
    full = lambda a: pl.BlockSpec(a.shape, lambda i: tuple(0 for _ in a.shape))
    return pl.pallas_call(
        _pack_body,
        grid=(NBLK,),
        in_specs=[
            pl.BlockSpec((DU, VB), lambda i: (0, i)),
            pl.BlockSpec((DI, VB), lambda i: (0, i)),
            full(wt),
            pl.BlockSpec((BB, FU), dense_i),
            pl.BlockSpec((BB, FI), dense_i),
            pl.BlockSpec((BB, FC), dense_i),
            full(wu), full(wi), full(wc),
            full(bu), full(bi), full(bc),
        ],
        out_specs=[pl.BlockSpec((LB, 128), lambda i: (i, 0))] * 8 + [
            pl.BlockSpec((NUM_TASKS, BB),
                         lambda i: (0, jnp.minimum(i, NDB - 1)))],
        out_shape=[jax.ShapeDtypeStruct((NLINES, 128), jnp.float32)] * 8 + [
            jax.ShapeDtypeStruct((NUM_TASKS, B), jnp.float32)],
    )(utabT, itabT, wt, uf, itf, cf, wu, wi, wc, bu, bi, bc)


def _sc_gather_body(pu0, pu1, pu2, pu3, pi0, pi1, pi2, pi3, uid, iid,
                    peT_out, ulv, ilv, gu, gi, pe, sem):
    wid = lax.axis_index("s") * NC + lax.axis_index("c")
    base = wid * BPW
    pltpu.sync_copy(uid.at[pl.ds(base, BPW)], ulv)
    pltpu.sync_copy(iid.at[pl.ds(base, BPW)], ilv)

    pus = (pu0, pu1, pu2, pu3)
    pis = (pi0, pi1, pi2, pi3)
    for t in range(NUM_TASKS):
        for c in range(BPW // IDX_CHUNK):
            s = pl.ds(c * IDX_CHUNK, IDX_CHUNK)
            pltpu.async_copy(pus[t].at[ulv.at[s]], gu.at[t, s], sem)
            pltpu.async_copy(pis[t].at[ilv.at[s]], gi.at[t, s], sem)
    for t in range(NUM_TASKS):
        pltpu.make_async_copy(pus[t].at[pl.ds(0, BPW)], gu.at[t], sem).wait()
        pltpu.make_async_copy(pis[t].at[pl.ds(0, BPW)], gi.at[t], sem).wait()

    def accum(k, carry):
        s = pl.ds(k * 16, 16)
        for t in range(NUM_TASKS):
            pe[t, s] = gu[t, s] + gi[t, s]
        return carry
    lax.fori_loop(0, BPW // 16, accum, 0)

    pltpu.sync_copy(pe, peT_out.at[:, pl.ds(base, BPW)])


_sc_gather = pl.kernel(
    _sc_gather_body,
    out_type=jax.ShapeDtypeStruct((NUM_TASKS, B), jnp.float32),
    mesh=plsc.VectorSubcoreMesh(core_axis_name="c", subcore_axis_name="s"),
    compiler_params=pltpu.CompilerParams(needs_layout_passes=False),
    scratch_types=[
        pltpu.VMEM((BPW,), jnp.int32),
        pltpu.VMEM((BPW,), jnp.int32),
        pltpu.VMEM((NUM_TASKS, BPW), jnp.float32),
        pltpu.VMEM((NUM_TASKS, BPW), jnp.float32),
        pltpu.VMEM((NUM_TASKS, BPW), jnp.float32),
        pltpu.SemaphoreType.DMA,
    ],
)


FB = 2048


def _final_body(accT, peT, bt, out):
    tn = (((0,), (0,)), ((), ()))
    eye = jnp.eye(NUM_TASKS, NUM_TASKS, dtype=jnp.float32)
    out[...] = lax.dot_general(
        accT[...] + peT[...], eye, tn,
        preferred_element_type=jnp.float32) + bt[...]


def _final_call(accT, peT, bt):
    full = lambda a: pl.BlockSpec(a.shape, lambda i: tuple(0 for _ in a.shape))
    return pl.pallas_call(
        _final_body,
        grid=(B // FB,),
        in_specs=[
            pl.BlockSpec((NUM_TASKS, FB), lambda i: (0, i)),
            pl.BlockSpec((NUM_TASKS, FB), lambda i: (0, i)),
            full(bt),
        ],
        out_specs=pl.BlockSpec((FB, NUM_TASKS), lambda i: (i, 0)),
        out_shape=jax.ShapeDtypeStruct((B, NUM_TASKS), jnp.float32),
    )(accT, peT, bt)


def kernel(user_id, user_features, item_id, item_features, cross_features,
           position, user_table, item_table, W_u, b_u, W_i, b_i, W_c, b_c,
           W_t, b_t):
    packed = _pack_call(
        user_table.T, item_table.T, W_t,
        user_features, item_features, cross_features,
        W_u, W_i, W_c,
        b_u.reshape(1, DU), b_i.reshape(1, DI), b_c.reshape(1, CROSS_OUT))
    lines, accT = packed[:8], packed[8]
    flats = [a.reshape(VFLAT) for a in lines]
    peT = _sc_gather(*flats, user_id, item_id)
    return _final_call(accT, peT, b_t.reshape(1, NUM_TASKS))
